# baseline (speedup 1.0000x reference)
import functools

import jax
import jax.numpy as jnp
from jax import lax
from jax.experimental import pallas as pl
from jax.experimental.pallas import tpu as pltpu
from jax.experimental.pallas import tpu_sc as plsc

_VOCAB = 1000000
_DIM = 32
_B = 4096
_C = 4096


@functools.lru_cache(maxsize=None)
def _make_sc_gather(V, D, B, C):
    NC, NS = 2, 16
    NW = NC * NS
    b_per_w = B // NW
    c_per_w = C // NW
    DH = D // 8
    NB = 8
    mesh = plsc.VectorSubcoreMesh(core_axis_name="c", subcore_axis_name="s")

    @functools.partial(
        pl.kernel,
        mesh=mesh,
        out_type=[
            jax.ShapeDtypeStruct((DH, 8, B), jnp.float32),
            jax.ShapeDtypeStruct((DH, 8, C), jnp.float32),
        ],
        scratch_types=[
            pltpu.VMEM((b_per_w + 2 * NB,), jnp.int32),
            pltpu.VMEM((b_per_w + 2 * NB,), jnp.int32),
            pltpu.VMEM((c_per_w + 2 * NB,), jnp.int32),
            pltpu.VMEM((c_per_w + 2 * NB,), jnp.int32),
            pltpu.VMEM((2, NB, DH, 8, 128), jnp.float32),
            pltpu.VMEM((DH, 8, b_per_w), jnp.float32),
            pltpu.VMEM((DH, 8, c_per_w), jnp.float32),
            pltpu.SemaphoreType.DMA,
            pltpu.SemaphoreType.DMA,
        ],
    )
    def gather_k(winT_hbm, ctcol_hbm, clane_hbm, woutT_hbm, xtcol_hbm,
                 xlane_hbm, outcT_hbm, outxT_hbm,
                 ctcol_v, clane_v, xtcol_v, xlane_v,
                 slots_v, cacc_v, xacc_v, sem0, sem1):
        wid = lax.axis_index("s") * NC + lax.axis_index("c")
        cbase = wid * b_per_w
        xbase = wid * c_per_w
        pltpu.sync_copy(ctcol_hbm.at[pl.ds(cbase, b_per_w)],
                        ctcol_v.at[pl.ds(0, b_per_w)])
        pltpu.sync_copy(clane_hbm.at[pl.ds(cbase, b_per_w)],
                        clane_v.at[pl.ds(0, b_per_w)])
        pltpu.sync_copy(xtcol_hbm.at[pl.ds(xbase, c_per_w)],
                        xtcol_v.at[pl.ds(0, c_per_w)])
        pltpu.sync_copy(xlane_hbm.at[pl.ds(xbase, c_per_w)],
                        xlane_v.at[pl.ds(0, c_per_w)])
        d16 = lax.iota(jnp.int32, 16)
        sems = (sem0, sem1)

        def gather_table(tab_hbm, tcol_v, lane_v, acc_v, n_words):
            nbatch = n_words // NB

            def fire(slot, sem, off16):
                tcol = tcol_v[pl.ds(off16, 16)]
                for b in range(NB):
                    pltpu.async_copy(
                        tab_hbm.at[:, :, pl.ds(pl.multiple_of(tcol[b], 128), 128)],
                        slots_v.at[slot, b], sem,
                    )

            def drain(slot, sem):
                for b in range(NB):
                    pltpu.make_async_copy(
                        tab_hbm.at[:, :, pl.ds(0, 128)],
                        slots_v.at[slot, b], sem,
                    ).wait()

            def extract(slot, p, off16, colg16):
                lane = lane_v[pl.ds(off16, 16)]
                for h in range(DH):
                    for s in range(8):
                        cur = acc_v[h, s, pl.ds(colg16, 16)]
                        for b in range(NB):
                            lb = lane[b]
                            lc16 = pl.multiple_of((lb // 16) * 16, 16)
                            li = jnp.broadcast_to(lb - lc16, (16,))
                            v = slots_v[slot, b, h, s, pl.ds(lc16, 16)]
                            gv = lax.gather(
                                v, li[:, None],
                                lax.GatherDimensionNumbers(
                                    offset_dims=(), collapsed_slice_dims=(0,),
                                    start_index_map=(0,)),
                                (1,),
                                mode=lax.GatherScatterMode.PROMISE_IN_BOUNDS)
                            cur = jnp.where(d16 == p * NB + b, gv, cur)
                        acc_v[h, s, pl.ds(colg16, 16)] = cur

            fire(0, sems[0], 0)
            fire(1, sems[1], NB)

            def body(u, _):
                colg16 = pl.multiple_of(u * 16, 16)
                for p in range(2):
                    t = 2 * u + p
                    drain(p, sems[p])
                    extract(p, p, t * NB, colg16)

                    @pl.when(t + 2 < nbatch)
                    def _():
                        fire(p, sems[p], (t + 2) * NB)
                return 0

            lax.fori_loop(0, nbatch // 2, body, 0)

        gather_table(winT_hbm, ctcol_v, clane_v, cacc_v, b_per_w)
        gather_table(woutT_hbm, xtcol_v, xlane_v, xacc_v, c_per_w)
        pltpu.sync_copy(cacc_v, outcT_hbm.at[:, :, pl.ds(cbase, b_per_w)])
        pltpu.sync_copy(xacc_v, outxT_hbm.at[:, :, pl.ds(xbase, c_per_w)])

    return gather_k


@functools.lru_cache(maxsize=None)
def _make_sc_gather_slice(V, D, B, nsl, isl):
    NC, NS = 2, 16
    NW = NC * NS
    n_per_w = B // NW
    DH = D // 8
    NB = 8
    mesh = plsc.VectorSubcoreMesh(core_axis_name="c", subcore_axis_name="s")

    @functools.partial(
        pl.kernel,
        mesh=mesh,
        out_type=jax.ShapeDtypeStruct((DH, 8, B // nsl), jnp.float32),
        scratch_types=[
            pltpu.VMEM((n_per_w + 2 * NB,), jnp.int32),
            pltpu.VMEM((n_per_w + 2 * NB,), jnp.int32),
            pltpu.VMEM((2, NB, DH, 8, 128), jnp.float32),
            pltpu.VMEM((DH, 8, n_per_w), jnp.float32),
            pltpu.SemaphoreType.DMA,
            pltpu.SemaphoreType.DMA,
        ],
    )
    def gather1_k(tab_hbm, tcol_hbm, lane_hbm, out_hbm,
                  tcol_v, lane_v, slots_v, acc_v, sem0, sem1):
        wid = lax.axis_index("s") * NC + lax.axis_index("c")

        @pl.when(wid % nsl == isl)
        def _active():
            base = wid * n_per_w
            obase = (wid // nsl) * n_per_w
            pltpu.sync_copy(tcol_hbm.at[pl.ds(base, n_per_w)],
                            tcol_v.at[pl.ds(0, n_per_w)])
            pltpu.sync_copy(lane_hbm.at[pl.ds(base, n_per_w)],
                            lane_v.at[pl.ds(0, n_per_w)])
            d16 = lax.iota(jnp.int32, 16)
            sems = (sem0, sem1)
            nbatch = n_per_w // NB

            def fire(slot, sem, off16):
                tcol = tcol_v[pl.ds(off16, 16)]
                for b in range(NB):
                    pltpu.async_copy(
                        tab_hbm.at[:, :, pl.ds(pl.multiple_of(tcol[b], 128), 128)],
                        slots_v.at[slot, b], sem,
                    )

            def drain(slot, sem):
                for b in range(NB):
                    pltpu.make_async_copy(
                        tab_hbm.at[:, :, pl.ds(0, 128)],
                        slots_v.at[slot, b], sem,
                    ).wait()

            def extract(slot, p, off16, colg16):
                lane = lane_v[pl.ds(off16, 16)]
                for h in range(DH):
                    for s in range(8):
                        cur = acc_v[h, s, pl.ds(colg16, 16)]
                        for b in range(NB):
                            lb = lane[b]
                            lc16 = pl.multiple_of((lb // 16) * 16, 16)
                            li = jnp.broadcast_to(lb - lc16, (16,))
                            v = slots_v[slot, b, h, s, pl.ds(lc16, 16)]
                            gv = lax.gather(
                                v, li[:, None],
                                lax.GatherDimensionNumbers(
                                    offset_dims=(), collapsed_slice_dims=(0,),
                                    start_index_map=(0,)),
                                (1,),
                                mode=lax.GatherScatterMode.PROMISE_IN_BOUNDS)
                            cur = jnp.where(d16 == p * NB + b, gv, cur)
                        acc_v[h, s, pl.ds(colg16, 16)] = cur

            fire(0, sems[0], 0)
            fire(1, sems[1], NB)

            def body(u, _):
                colg16 = pl.multiple_of(u * 16, 16)
                for p in range(2):
                    t = 2 * u + p
                    drain(p, sems[p])
                    extract(p, p, t * NB, colg16)

                    @pl.when(t + 2 < nbatch)
                    def _():
                        fire(p, sems[p], (t + 2) * NB)
                return 0

            lax.fori_loop(0, nbatch // 2, body, 0)
            pltpu.sync_copy(acc_v, out_hbm.at[:, :, pl.ds(obase, n_per_w)])

    return gather1_k


def _mm_body(cvT_ref, xvT_ref, out_ref):
    out_ref[...] = lax.dot_general(
        cvT_ref[...], xvT_ref[...],
        (((0,), (0,)), ((), ())),
        preferred_element_type=jnp.float32,
    )


def _mm_body_alias(cvT_ref, xvT_ref, prev_ref, out_ref):
    del prev_ref
    out_ref[...] = lax.dot_general(
        cvT_ref[...], xvT_ref[...],
        (((0,), (0,)), ((), ())),
        preferred_element_type=jnp.float32,
    )


_K = 4
_BM = 128


def _mm_slice(cvT_i, xvT, prev, isl):
    rows = _B // _K
    grid = (rows // _BM, 1)
    out_spec = pl.BlockSpec((_BM, _C), lambda s, j: (_K * s + isl, j))
    in_specs = [
        pl.BlockSpec((_DIM, _BM), lambda s, j: (0, s)),
        pl.BlockSpec((_DIM, _C), lambda s, j: (0, j)),
    ]
    if prev is None:
        return pl.pallas_call(
            _mm_body,
            grid=grid,
            in_specs=in_specs,
            out_specs=out_spec,
            out_shape=jax.ShapeDtypeStruct((_B, _C), jnp.float32),
        )(cvT_i, xvT)
    in_specs.append(pl.BlockSpec(memory_space=pltpu.MemorySpace.HBM))
    return pl.pallas_call(
        _mm_body_alias,
        grid=grid,
        in_specs=in_specs,
        out_specs=out_spec,
        out_shape=jax.ShapeDtypeStruct((_B, _C), jnp.float32),
        input_output_aliases={2: 0},
    )(cvT_i, xvT, prev)


def kernel(center_words, all_context_words, W_in, W_out):
    cidx = center_words.astype(jnp.int32)
    xidx = all_context_words.astype(jnp.int32)
    ctcol = (cidx // 128) * 128
    clane = cidx % 128
    xtcol = (xidx // 128) * 128
    xlane = xidx % 128
    winT = W_in.T.reshape(_DIM // 8, 8, _VOCAB)
    woutT = W_out.T.reshape(_DIM // 8, 8, _VOCAB)
    xvT = _make_sc_gather_slice(_VOCAB, _DIM, _C, 1, 0)(
        woutT, xtcol, xlane).reshape(_DIM, _C)
    cv_slices = []
    for i in range(_K):
        cvi = _make_sc_gather_slice(_VOCAB, _DIM, _B, _K, i)(
            winT, ctcol, clane)
        cv_slices.append(cvi.reshape(_DIM, _B // _K))
    out = None
    for i in range(_K):
        out = _mm_slice(cv_slices[i], xvT, out, i)
    return out


# speedup vs baseline: 1.2867x; 1.2867x over previous
import functools

import jax
import jax.numpy as jnp
from jax import lax
from jax.experimental import pallas as pl
from jax.experimental.pallas import tpu as pltpu
from jax.experimental.pallas import tpu_sc as plsc

_VOCAB = 1000000
_DIM = 32
_B = 4096
_C = 4096


@functools.lru_cache(maxsize=None)
def _make_sc_gather(V, D, B, C):
    NC, NS = 2, 16
    NW = NC * NS
    b_per_w = B // NW
    c_per_w = C // NW
    DH = D // 8
    NB = 8
    mesh = plsc.VectorSubcoreMesh(core_axis_name="c", subcore_axis_name="s")

    @functools.partial(
        pl.kernel,
        mesh=mesh,
        out_type=[
            jax.ShapeDtypeStruct((DH, 8, B), jnp.float32),
            jax.ShapeDtypeStruct((DH, 8, C), jnp.float32),
        ],
        scratch_types=[
            pltpu.VMEM((b_per_w + 2 * NB,), jnp.int32),
            pltpu.VMEM((b_per_w + 2 * NB,), jnp.int32),
            pltpu.VMEM((c_per_w + 2 * NB,), jnp.int32),
            pltpu.VMEM((c_per_w + 2 * NB,), jnp.int32),
            pltpu.VMEM((2, NB, DH, 8, 128), jnp.float32),
            pltpu.VMEM((DH, 8, b_per_w), jnp.float32),
            pltpu.VMEM((DH, 8, c_per_w), jnp.float32),
            pltpu.SemaphoreType.DMA,
            pltpu.SemaphoreType.DMA,
        ],
    )
    def gather_k(winT_hbm, ctcol_hbm, clane_hbm, woutT_hbm, xtcol_hbm,
                 xlane_hbm, outcT_hbm, outxT_hbm,
                 ctcol_v, clane_v, xtcol_v, xlane_v,
                 slots_v, cacc_v, xacc_v, sem0, sem1):
        wid = lax.axis_index("s") * NC + lax.axis_index("c")
        cbase = wid * b_per_w
        xbase = wid * c_per_w
        pltpu.sync_copy(ctcol_hbm.at[pl.ds(cbase, b_per_w)],
                        ctcol_v.at[pl.ds(0, b_per_w)])
        pltpu.sync_copy(clane_hbm.at[pl.ds(cbase, b_per_w)],
                        clane_v.at[pl.ds(0, b_per_w)])
        pltpu.sync_copy(xtcol_hbm.at[pl.ds(xbase, c_per_w)],
                        xtcol_v.at[pl.ds(0, c_per_w)])
        pltpu.sync_copy(xlane_hbm.at[pl.ds(xbase, c_per_w)],
                        xlane_v.at[pl.ds(0, c_per_w)])
        d16 = lax.iota(jnp.int32, 16)
        sems = (sem0, sem1)

        def gather_table(tab_hbm, tcol_v, lane_v, acc_v, n_words):
            nbatch = n_words // NB

            def fire(slot, sem, off16):
                tcol = tcol_v[pl.ds(off16, 16)]
                for b in range(NB):
                    pltpu.async_copy(
                        tab_hbm.at[:, :, pl.ds(pl.multiple_of(tcol[b], 128), 128)],
                        slots_v.at[slot, b], sem,
                    )

            def drain(slot, sem):
                for b in range(NB):
                    pltpu.make_async_copy(
                        tab_hbm.at[:, :, pl.ds(0, 128)],
                        slots_v.at[slot, b], sem,
                    ).wait()

            def extract(slot, p, off16, colg16):
                lane = lane_v[pl.ds(off16, 16)]
                for h in range(DH):
                    for s in range(8):
                        cur = acc_v[h, s, pl.ds(colg16, 16)]
                        for b in range(NB):
                            lb = lane[b]
                            lc16 = pl.multiple_of((lb // 16) * 16, 16)
                            li = jnp.broadcast_to(lb - lc16, (16,))
                            v = slots_v[slot, b, h, s, pl.ds(lc16, 16)]
                            gv = lax.gather(
                                v, li[:, None],
                                lax.GatherDimensionNumbers(
                                    offset_dims=(), collapsed_slice_dims=(0,),
                                    start_index_map=(0,)),
                                (1,),
                                mode=lax.GatherScatterMode.PROMISE_IN_BOUNDS)
                            cur = jnp.where(d16 == p * NB + b, gv, cur)
                        acc_v[h, s, pl.ds(colg16, 16)] = cur

            fire(0, sems[0], 0)
            fire(1, sems[1], NB)

            def body(u, _):
                colg16 = pl.multiple_of(u * 16, 16)
                for p in range(2):
                    t = 2 * u + p
                    drain(p, sems[p])
                    extract(p, p, t * NB, colg16)

                    @pl.when(t + 2 < nbatch)
                    def _():
                        fire(p, sems[p], (t + 2) * NB)
                return 0

            lax.fori_loop(0, nbatch // 2, body, 0)

        gather_table(winT_hbm, ctcol_v, clane_v, cacc_v, b_per_w)
        gather_table(woutT_hbm, xtcol_v, xlane_v, xacc_v, c_per_w)
        pltpu.sync_copy(cacc_v, outcT_hbm.at[:, :, pl.ds(cbase, b_per_w)])
        pltpu.sync_copy(xacc_v, outxT_hbm.at[:, :, pl.ds(xbase, c_per_w)])

    return gather_k


@functools.lru_cache(maxsize=None)
def _make_sc_gather1(V, D, N):
    NC, NS = 2, 16
    NW = NC * NS
    n_per_w = N // NW
    DH = D // 8
    NB = 8
    mesh = plsc.VectorSubcoreMesh(core_axis_name="c", subcore_axis_name="s")

    @functools.partial(
        pl.kernel,
        mesh=mesh,
        out_type=jax.ShapeDtypeStruct((DH, 8, N), jnp.float32),
        scratch_types=[
            pltpu.VMEM((n_per_w + 2 * NB,), jnp.int32),
            pltpu.VMEM((n_per_w + 2 * NB,), jnp.int32),
            pltpu.VMEM((2, NB, DH, 8, 128), jnp.float32),
            pltpu.VMEM((DH, 8, n_per_w), jnp.float32),
            pltpu.SemaphoreType.DMA,
            pltpu.SemaphoreType.DMA,
        ],
    )
    def gather1_k(tab_hbm, tcol_hbm, lane_hbm, out_hbm,
                  tcol_v, lane_v, slots_v, acc_v, sem0, sem1):
        wid = lax.axis_index("s") * NC + lax.axis_index("c")
        base = wid * n_per_w
        pltpu.sync_copy(tcol_hbm.at[pl.ds(base, n_per_w)],
                        tcol_v.at[pl.ds(0, n_per_w)])
        pltpu.sync_copy(lane_hbm.at[pl.ds(base, n_per_w)],
                        lane_v.at[pl.ds(0, n_per_w)])
        d16 = lax.iota(jnp.int32, 16)
        sems = (sem0, sem1)
        nbatch = n_per_w // NB

        def fire(slot, sem, off16):
            tcol = tcol_v[pl.ds(off16, 16)]
            for b in range(NB):
                pltpu.async_copy(
                    tab_hbm.at[:, :, pl.ds(pl.multiple_of(tcol[b], 128), 128)],
                    slots_v.at[slot, b], sem,
                )

        def drain(slot, sem):
            for b in range(NB):
                pltpu.make_async_copy(
                    tab_hbm.at[:, :, pl.ds(0, 128)],
                    slots_v.at[slot, b], sem,
                ).wait()

        def extract(slot, p, off16, colg16):
            lane = lane_v[pl.ds(off16, 16)]
            for h in range(DH):
                for s in range(8):
                    cur = acc_v[h, s, pl.ds(colg16, 16)]
                    for b in range(NB):
                        lb = lane[b]
                        lc16 = pl.multiple_of((lb // 16) * 16, 16)
                        li = jnp.broadcast_to(lb - lc16, (16,))
                        v = slots_v[slot, b, h, s, pl.ds(lc16, 16)]
                        gv = lax.gather(
                            v, li[:, None],
                            lax.GatherDimensionNumbers(
                                offset_dims=(), collapsed_slice_dims=(0,),
                                start_index_map=(0,)),
                            (1,),
                            mode=lax.GatherScatterMode.PROMISE_IN_BOUNDS)
                        cur = jnp.where(d16 == p * NB + b, gv, cur)
                    acc_v[h, s, pl.ds(colg16, 16)] = cur

        fire(0, sems[0], 0)
        fire(1, sems[1], NB)

        def body(u, _):
            colg16 = pl.multiple_of(u * 16, 16)
            for p in range(2):
                t = 2 * u + p
                drain(p, sems[p])
                extract(p, p, t * NB, colg16)

                @pl.when(t + 2 < nbatch)
                def _():
                    fire(p, sems[p], (t + 2) * NB)
            return 0

        lax.fori_loop(0, nbatch // 2, body, 0)
        pltpu.sync_copy(acc_v, out_hbm.at[:, :, pl.ds(base, n_per_w)])

    return gather1_k


_W = 32


@functools.lru_cache(maxsize=None)
def _make_tc_gather(V, D, N):
    DH = D // 8
    nsteps = N // _W

    def body(tcb_ref, lane_ref, *refs):
        out_ref = refs[-1]
        blocks = refs[:-1]
        i = pl.program_id(0)
        rows = []
        for j in range(_W):
            l = lane_ref[i * _W + j]
            blk = jnp.reshape(blocks[j][...], (D, 128))
            m = (lax.broadcasted_iota(jnp.int32, (1, 128), 1) == l
                 ).astype(jnp.float32)
            rows.append(lax.dot_general(
                m, blk, (((1,), (1,)), ((), ())),
                preferred_element_type=jnp.float32))
        out_ref[...] = jnp.concatenate(rows, axis=0)[None]

    def table_spec(j):
        return pl.BlockSpec(
            (DH, 8, 128), lambda i, tcb, lane: (0, 0, tcb[i * _W + j]))

    grid_spec = pltpu.PrefetchScalarGridSpec(
        num_scalar_prefetch=2,
        grid=(nsteps,),
        in_specs=[table_spec(j) for j in range(_W)],
        out_specs=pl.BlockSpec((1, _W, D), lambda i, tcb, lane: (i, 0, 0)),
    )
    return pl.pallas_call(
        body,
        grid_spec=grid_spec,
        out_shape=jax.ShapeDtypeStruct((nsteps, _W, D), jnp.float32),
    )


def _mm_body(cvT_ref, xv_ref, out_ref):
    out_ref[...] = lax.dot_general(
        cvT_ref[...], xv_ref[...],
        (((0,), (1,)), ((), ())),
        preferred_element_type=jnp.float32,
    )


def _matmul(cvT, xv):
    BM = 512
    grid = (_B // BM, 1)
    return pl.pallas_call(
        _mm_body,
        grid=grid,
        in_specs=[
            pl.BlockSpec((_DIM, BM), lambda i, j: (0, i)),
            pl.BlockSpec((_C, _DIM), lambda i, j: (0, 0)),
        ],
        out_specs=pl.BlockSpec((BM, _C), lambda i, j: (i, j)),
        out_shape=jax.ShapeDtypeStruct((_B, _C), jnp.float32),
    )(cvT, xv)


def kernel(center_words, all_context_words, W_in, W_out):
    cidx = center_words.astype(jnp.int32)
    xidx = all_context_words.astype(jnp.int32)
    ctcol = (cidx // 128) * 128
    clane = cidx % 128
    xtcb = xidx // 128
    xlane = xidx % 128
    winT = W_in.T.reshape(_DIM // 8, 8, _VOCAB)
    woutT = W_out.T.reshape(_DIM // 8, 8, _VOCAB)
    cvT = _make_sc_gather1(_VOCAB, _DIM, _B)(
        winT, ctcol, clane).reshape(_DIM, _B)
    xv = _make_tc_gather(_VOCAB, _DIM, _C)(
        xtcb, xlane, *([woutT] * _W)).reshape(_C, _DIM)
    return _matmul(cvT, xv)


# speedup vs baseline: 1.4358x; 1.1159x over previous
import functools

import jax
import jax.numpy as jnp
from jax import lax
from jax.experimental import pallas as pl
from jax.experimental.pallas import tpu as pltpu
from jax.experimental.pallas import tpu_sc as plsc

_VOCAB = 1000000
_DIM = 32
_B = 4096
_C = 4096


@functools.lru_cache(maxsize=None)
def _make_sc_gather(V, D, B, C):
    NC, NS = 2, 16
    NW = NC * NS
    b_per_w = B // NW
    c_per_w = C // NW
    DH = D // 8
    RING = 16
    mesh = plsc.VectorSubcoreMesh(core_axis_name="c", subcore_axis_name="s")

    @functools.partial(
        pl.kernel,
        mesh=mesh,
        out_type=[
            jax.ShapeDtypeStruct((DH, 8, B), jnp.float32),
            jax.ShapeDtypeStruct((DH, 8, C), jnp.float32),
        ],
        scratch_types=[
            pltpu.VMEM((b_per_w,), jnp.int32),
            pltpu.VMEM((b_per_w,), jnp.int32),
            pltpu.VMEM((c_per_w,), jnp.int32),
            pltpu.VMEM((c_per_w,), jnp.int32),
            pltpu.VMEM((RING, DH, 8, 128), jnp.float32),
            pltpu.VMEM((DH, 8, b_per_w), jnp.float32),
            pltpu.VMEM((DH, 8, c_per_w), jnp.float32),
            pltpu.SemaphoreType.DMA((RING,)),
        ],
    )
    def gather_k(winT_hbm, ctcol_hbm, clane_hbm, woutT_hbm, xtcol_hbm,
                 xlane_hbm, outcT_hbm, outxT_hbm,
                 ctcol_v, clane_v, xtcol_v, xlane_v,
                 slots_v, cacc_v, xacc_v, sems):
        wid = lax.axis_index("s") * NC + lax.axis_index("c")
        cbase = wid * b_per_w
        xbase = wid * c_per_w
        pltpu.sync_copy(ctcol_hbm.at[pl.ds(cbase, b_per_w)], ctcol_v)
        pltpu.sync_copy(clane_hbm.at[pl.ds(cbase, b_per_w)], clane_v)
        pltpu.sync_copy(xtcol_hbm.at[pl.ds(xbase, c_per_w)], xtcol_v)
        pltpu.sync_copy(xlane_hbm.at[pl.ds(xbase, c_per_w)], xlane_v)
        d16 = lax.iota(jnp.int32, 16)

        def gather_table(tab_hbm, tcol_v, lane_v, acc_v, n_words):
            ngrp = n_words // RING

            def fire(off16):
                tcol = tcol_v[pl.ds(off16, 16)]
                for b in range(RING):
                    pltpu.async_copy(
                        tab_hbm.at[:, :, pl.ds(pl.multiple_of(tcol[b], 128), 128)],
                        slots_v.at[b], sems.at[b],
                    )

            fire(0)

            def body(u, _):
                colg16 = pl.multiple_of(u * 16, 16)
                lane = lane_v[pl.ds(colg16, 16)]

                for b in range(RING):
                    pltpu.make_async_copy(
                        tab_hbm.at[:, :, pl.ds(0, 128)],
                        slots_v.at[b], sems.at[b],
                    ).wait()
                    lb = lane[b]
                    lc16 = pl.multiple_of((lb // 16) * 16, 16)
                    li = jnp.broadcast_to(lb - lc16, (16,))
                    for h in range(DH):
                        for s in range(8):
                            v = slots_v[b, h, s, pl.ds(lc16, 16)]
                            gv = lax.gather(
                                v, li[:, None],
                                lax.GatherDimensionNumbers(
                                    offset_dims=(), collapsed_slice_dims=(0,),
                                    start_index_map=(0,)),
                                (1,),
                                mode=lax.GatherScatterMode.PROMISE_IN_BOUNDS)
                            cur = acc_v[h, s, pl.ds(colg16, 16)]
                            acc_v[h, s, pl.ds(colg16, 16)] = (
                                jnp.where(d16 == b, gv, cur))

                    @pl.when(u + 1 < ngrp)
                    def _refire():
                        tcoln = tcol_v[pl.ds(pl.multiple_of((u + 1) * 16, 16), 16)]
                        pltpu.async_copy(
                            tab_hbm.at[:, :, pl.ds(
                                pl.multiple_of(tcoln[b], 128), 128)],
                            slots_v.at[b], sems.at[b],
                        )
                return 0

            lax.fori_loop(0, ngrp, body, 0)

        gather_table(winT_hbm, ctcol_v, clane_v, cacc_v, b_per_w)
        gather_table(woutT_hbm, xtcol_v, xlane_v, xacc_v, c_per_w)
        pltpu.sync_copy(cacc_v, outcT_hbm.at[:, :, pl.ds(cbase, b_per_w)])
        pltpu.sync_copy(xacc_v, outxT_hbm.at[:, :, pl.ds(xbase, c_per_w)])

    return gather_k


def _mm_body(cvT_ref, xvT_ref, out_ref):
    out_ref[...] = lax.dot_general(
        cvT_ref[...], xvT_ref[...],
        (((0,), (0,)), ((), ())),
        preferred_element_type=jnp.float32,
    )


def _matmul(cvT, xvT):
    BM = 512
    BN = 4096
    grid = (_B // BM, _C // BN)
    return pl.pallas_call(
        _mm_body,
        grid=grid,
        in_specs=[
            pl.BlockSpec((_DIM, BM), lambda i, j: (0, i)),
            pl.BlockSpec((_DIM, BN), lambda i, j: (0, j)),
        ],
        out_specs=pl.BlockSpec((BM, BN), lambda i, j: (i, j)),
        out_shape=jax.ShapeDtypeStruct((_B, _C), jnp.float32),
    )(cvT, xvT)


def kernel(center_words, all_context_words, W_in, W_out):
    cidx = center_words.astype(jnp.int32)
    xidx = all_context_words.astype(jnp.int32)
    ctcol = (cidx // 128) * 128
    clane = cidx % 128
    xtcol = (xidx // 128) * 128
    xlane = xidx % 128
    cvT3, xvT3 = _make_sc_gather(_VOCAB, _DIM, _B, _C)(
        W_in.T.reshape(_DIM // 8, 8, _VOCAB), ctcol, clane,
        W_out.T.reshape(_DIM // 8, 8, _VOCAB), xtcol, xlane)
    cvT = cvT3.reshape(_DIM, _B)
    xvT = xvT3.reshape(_DIM, _C)
    return _matmul(cvT, xvT)


# speedup vs baseline: 1.8849x; 1.3127x over previous
import functools

import jax
import jax.numpy as jnp
from jax import lax
from jax.experimental import pallas as pl
from jax.experimental.pallas import tpu as pltpu
from jax.experimental.pallas import tpu_sc as plsc

_VOCAB = 1000000
_DIM = 32
_B = 4096
_C = 4096


@functools.lru_cache(maxsize=None)
def _make_sc_gather(V, D, B, C):
    NC, NS = 2, 16
    NW = NC * NS
    b_per_w = B // NW
    c_per_w = C // NW
    DH = D // 8
    NB = 8
    mesh = plsc.VectorSubcoreMesh(core_axis_name="c", subcore_axis_name="s")

    @functools.partial(
        pl.kernel,
        mesh=mesh,
        out_type=[
            jax.ShapeDtypeStruct((DH, 8, B), jnp.float32),
            jax.ShapeDtypeStruct((DH, 8, C), jnp.float32),
        ],
        scratch_types=[
            pltpu.VMEM((b_per_w + 2 * NB,), jnp.int32),
            pltpu.VMEM((b_per_w + 2 * NB,), jnp.int32),
            pltpu.VMEM((c_per_w + 2 * NB,), jnp.int32),
            pltpu.VMEM((c_per_w + 2 * NB,), jnp.int32),
            pltpu.VMEM((2, NB, DH, 8, 128), jnp.float32),
            pltpu.VMEM((DH, 8, b_per_w), jnp.float32),
            pltpu.VMEM((DH, 8, c_per_w), jnp.float32),
            pltpu.SemaphoreType.DMA,
            pltpu.SemaphoreType.DMA,
        ],
    )
    def gather_k(winT_hbm, ctcol_hbm, clane_hbm, woutT_hbm, xtcol_hbm,
                 xlane_hbm, outcT_hbm, outxT_hbm,
                 ctcol_v, clane_v, xtcol_v, xlane_v,
                 slots_v, cacc_v, xacc_v, sem0, sem1):
        wid = lax.axis_index("s") * NC + lax.axis_index("c")
        cbase = wid * b_per_w
        xbase = wid * c_per_w
        pltpu.sync_copy(ctcol_hbm.at[pl.ds(cbase, b_per_w)],
                        ctcol_v.at[pl.ds(0, b_per_w)])
        pltpu.sync_copy(clane_hbm.at[pl.ds(cbase, b_per_w)],
                        clane_v.at[pl.ds(0, b_per_w)])
        pltpu.sync_copy(xtcol_hbm.at[pl.ds(xbase, c_per_w)],
                        xtcol_v.at[pl.ds(0, c_per_w)])
        pltpu.sync_copy(xlane_hbm.at[pl.ds(xbase, c_per_w)],
                        xlane_v.at[pl.ds(0, c_per_w)])
        d16 = lax.iota(jnp.int32, 16)
        sems = (sem0, sem1)

        def gather_table(tab_hbm, tcol_v, lane_v, acc_v, n_words):
            nbatch = n_words // NB

            def fire(slot, sem, off16):
                tcol = tcol_v[pl.ds(off16, 16)]
                for b in range(NB):
                    pltpu.async_copy(
                        tab_hbm.at[:, :, pl.ds(pl.multiple_of(tcol[b], 128), 128)],
                        slots_v.at[slot, b], sem,
                    )

            def drain(slot, sem):
                for b in range(NB):
                    pltpu.make_async_copy(
                        tab_hbm.at[:, :, pl.ds(0, 128)],
                        slots_v.at[slot, b], sem,
                    ).wait()

            def extract(slot, p, off16, colg16):
                lane = lane_v[pl.ds(off16, 16)]
                for h in range(DH):
                    for s in range(8):
                        cur = acc_v[h, s, pl.ds(colg16, 16)]
                        for b in range(NB):
                            lb = lane[b]
                            lc16 = pl.multiple_of((lb // 16) * 16, 16)
                            li = jnp.broadcast_to(lb - lc16, (16,))
                            v = slots_v[slot, b, h, s, pl.ds(lc16, 16)]
                            gv = lax.gather(
                                v, li[:, None],
                                lax.GatherDimensionNumbers(
                                    offset_dims=(), collapsed_slice_dims=(0,),
                                    start_index_map=(0,)),
                                (1,),
                                mode=lax.GatherScatterMode.PROMISE_IN_BOUNDS)
                            cur = jnp.where(d16 == p * NB + b, gv, cur)
                        acc_v[h, s, pl.ds(colg16, 16)] = cur

            fire(0, sems[0], 0)
            fire(1, sems[1], NB)

            def body(u, _):
                colg16 = pl.multiple_of(u * 16, 16)
                for p in range(2):
                    t = 2 * u + p
                    drain(p, sems[p])
                    extract(p, p, t * NB, colg16)

                    @pl.when(t + 2 < nbatch)
                    def _():
                        fire(p, sems[p], (t + 2) * NB)
                return 0

            lax.fori_loop(0, nbatch // 2, body, 0)

        gather_table(winT_hbm, ctcol_v, clane_v, cacc_v, b_per_w)
        gather_table(woutT_hbm, xtcol_v, xlane_v, xacc_v, c_per_w)
        pltpu.sync_copy(cacc_v, outcT_hbm.at[:, :, pl.ds(cbase, b_per_w)])
        pltpu.sync_copy(xacc_v, outxT_hbm.at[:, :, pl.ds(xbase, c_per_w)])

    return gather_k


def _mm_body(cvT_ref, xvT_ref, out_ref):
    out_ref[...] = lax.dot_general(
        cvT_ref[...], xvT_ref[...],
        (((0,), (0,)), ((), ())),
        preferred_element_type=jnp.float32,
    )


def _matmul(cvT, xvT):
    BM = 512
    BN = 4096
    grid = (_B // BM, _C // BN)
    return pl.pallas_call(
        _mm_body,
        grid=grid,
        in_specs=[
            pl.BlockSpec((_DIM, BM), lambda i, j: (0, i)),
            pl.BlockSpec((_DIM, BN), lambda i, j: (0, j)),
        ],
        out_specs=pl.BlockSpec((BM, BN), lambda i, j: (i, j)),
        out_shape=jax.ShapeDtypeStruct((_B, _C), jnp.float32),
    )(cvT, xvT)


def kernel(center_words, all_context_words, W_in, W_out):
    cidx = center_words.astype(jnp.int32)
    xidx = all_context_words.astype(jnp.int32)
    ctcol = (cidx // 128) * 128
    clane = cidx % 128
    xtcol = (xidx // 128) * 128
    xlane = xidx % 128
    cvT3, xvT3 = _make_sc_gather(_VOCAB, _DIM, _B, _C)(
        W_in.T.reshape(_DIM // 8, 8, _VOCAB), ctcol, clane,
        W_out.T.reshape(_DIM // 8, 8, _VOCAB), xtcol, xlane)
    cvT = cvT3.reshape(_DIM, _B)
    xvT = xvT3.reshape(_DIM, _C)
    return _matmul(cvT, xvT)


# speedup vs baseline: 1.9766x; 1.0487x over previous
import functools

import jax
import jax.numpy as jnp
from jax import lax
from jax.experimental import pallas as pl
from jax.experimental.pallas import tpu as pltpu
from jax.experimental.pallas import tpu_sc as plsc

_VOCAB = 1000000
_DIM = 32
_B = 4096
_C = 4096


@functools.lru_cache(maxsize=None)
def _make_sc_gather(V, D, B, C):
    NC, NS = 2, 16
    NW = NC * NS
    b_per_w = B // NW
    c_per_w = C // NW
    DH = D // 8
    NB = 4
    NBUF = 4
    mesh = plsc.VectorSubcoreMesh(core_axis_name="c", subcore_axis_name="s")

    @functools.partial(
        pl.kernel,
        mesh=mesh,
        out_type=[
            jax.ShapeDtypeStruct((DH, 8, B), jnp.float32),
            jax.ShapeDtypeStruct((DH, 8, C), jnp.float32),
        ],
        scratch_types=[
            pltpu.VMEM((b_per_w + 2 * NB,), jnp.int32),
            pltpu.VMEM((b_per_w + 2 * NB,), jnp.int32),
            pltpu.VMEM((c_per_w + 2 * NB,), jnp.int32),
            pltpu.VMEM((c_per_w + 2 * NB,), jnp.int32),
            pltpu.VMEM((NBUF, NB, DH, 8, 128), jnp.float32),
            pltpu.VMEM((DH, 8, b_per_w), jnp.float32),
            pltpu.VMEM((DH, 8, c_per_w), jnp.float32),
            pltpu.SemaphoreType.DMA,
            pltpu.SemaphoreType.DMA,
            pltpu.SemaphoreType.DMA,
            pltpu.SemaphoreType.DMA,
        ],
    )
    def gather_k(winT_hbm, ctcol_hbm, clane_hbm, woutT_hbm, xtcol_hbm,
                 xlane_hbm, outcT_hbm, outxT_hbm,
                 ctcol_v, clane_v, xtcol_v, xlane_v,
                 slots_v, cacc_v, xacc_v, sem0, sem1, sem2, sem3):
        wid = lax.axis_index("s") * NC + lax.axis_index("c")
        cbase = wid * b_per_w
        xbase = wid * c_per_w
        pltpu.sync_copy(ctcol_hbm.at[pl.ds(cbase, b_per_w)],
                        ctcol_v.at[pl.ds(0, b_per_w)])
        pltpu.sync_copy(clane_hbm.at[pl.ds(cbase, b_per_w)],
                        clane_v.at[pl.ds(0, b_per_w)])
        pltpu.sync_copy(xtcol_hbm.at[pl.ds(xbase, c_per_w)],
                        xtcol_v.at[pl.ds(0, c_per_w)])
        pltpu.sync_copy(xlane_hbm.at[pl.ds(xbase, c_per_w)],
                        xlane_v.at[pl.ds(0, c_per_w)])
        d16 = lax.iota(jnp.int32, 16)
        sems = (sem0, sem1, sem2, sem3)

        def gather_table(tab_hbm, tcol_v, lane_v, acc_v, n_words):
            nbatch = n_words // NB

            def fire(slot, sem, off16):
                tcol = tcol_v[pl.ds(off16, 16)]
                for b in range(NB):
                    pltpu.async_copy(
                        tab_hbm.at[:, :, pl.ds(pl.multiple_of(tcol[b], 128), 128)],
                        slots_v.at[slot, b], sem,
                    )

            def drain(slot, sem):
                for b in range(NB):
                    pltpu.make_async_copy(
                        tab_hbm.at[:, :, pl.ds(0, 128)],
                        slots_v.at[slot, b], sem,
                    ).wait()

            def extract(slot, p, off16, colg16):
                lane = lane_v[pl.ds(off16, 16)]
                for h in range(DH):
                    for s in range(8):
                        cur = acc_v[h, s, pl.ds(colg16, 16)]
                        for b in range(NB):
                            lb = lane[b]
                            lc16 = pl.multiple_of((lb // 16) * 16, 16)
                            li = jnp.broadcast_to(lb - lc16, (16,))
                            v = slots_v[slot, b, h, s, pl.ds(lc16, 16)]
                            gv = lax.gather(
                                v, li[:, None],
                                lax.GatherDimensionNumbers(
                                    offset_dims=(), collapsed_slice_dims=(0,),
                                    start_index_map=(0,)),
                                (1,),
                                mode=lax.GatherScatterMode.PROMISE_IN_BOUNDS)
                            cur = jnp.where(d16 == p * NB + b, gv, cur)
                        acc_v[h, s, pl.ds(colg16, 16)] = cur

            for q in range(NBUF):
                fire(q, sems[q], q * NB)

            def body(u, _):
                colg16 = pl.multiple_of(u * 16, 16)
                for p in range(NBUF):
                    t = NBUF * u + p
                    drain(p, sems[p])
                    extract(p, p, t * NB, colg16)

                    @pl.when(t + NBUF < nbatch)
                    def _():
                        fire(p, sems[p], (t + NBUF) * NB)
                return 0

            lax.fori_loop(0, nbatch // NBUF, body, 0)

        gather_table(winT_hbm, ctcol_v, clane_v, cacc_v, b_per_w)
        gather_table(woutT_hbm, xtcol_v, xlane_v, xacc_v, c_per_w)
        pltpu.sync_copy(cacc_v, outcT_hbm.at[:, :, pl.ds(cbase, b_per_w)])
        pltpu.sync_copy(xacc_v, outxT_hbm.at[:, :, pl.ds(xbase, c_per_w)])

    return gather_k


def _mm_body(cvT_ref, xvT_ref, out_ref):
    out_ref[...] = lax.dot_general(
        cvT_ref[...], xvT_ref[...],
        (((0,), (0,)), ((), ())),
        preferred_element_type=jnp.float32,
    )


def _matmul(cvT, xvT):
    BM = 512
    BN = 4096
    grid = (_B // BM, _C // BN)
    return pl.pallas_call(
        _mm_body,
        grid=grid,
        in_specs=[
            pl.BlockSpec((_DIM, BM), lambda i, j: (0, i)),
            pl.BlockSpec((_DIM, BN), lambda i, j: (0, j)),
        ],
        out_specs=pl.BlockSpec((BM, BN), lambda i, j: (i, j)),
        out_shape=jax.ShapeDtypeStruct((_B, _C), jnp.float32),
    )(cvT, xvT)


def kernel(center_words, all_context_words, W_in, W_out):
    cidx = center_words.astype(jnp.int32)
    xidx = all_context_words.astype(jnp.int32)
    ctcol = (cidx // 128) * 128
    clane = cidx % 128
    xtcol = (xidx // 128) * 128
    xlane = xidx % 128
    cvT3, xvT3 = _make_sc_gather(_VOCAB, _DIM, _B, _C)(
        W_in.T.reshape(_DIM // 8, 8, _VOCAB), ctcol, clane,
        W_out.T.reshape(_DIM // 8, 8, _VOCAB), xtcol, xlane)
    cvT = cvT3.reshape(_DIM, _B)
    xvT = xvT3.reshape(_DIM, _C)
    return _matmul(cvT, xvT)
